# Initial kernel scaffold; baseline (speedup 1.0000x reference)
import jax
import jax.numpy as jnp
from jax.experimental import pallas as pl


def kernel(hidden_states, position_ids, Wq, Wk, Wv, Wo):
    raise NotImplementedError("write your pallas kernel here")


# speedup vs baseline: 1.7597x; 1.7597x over previous
import jax
import jax.numpy as jnp
import numpy as np
from jax.experimental import pallas as pl
from jax.experimental.pallas import tpu as pltpu

B, S, HID = 1, 2048, 2048
NH, NKV, HD = 16, 4, 128
THETA = 10000.0
N_REP = NH // NKV
HALF = HD // 2
SCALE = 1.0 / np.sqrt(HD)

BS = 512
BQ = 512
BK = 512


def _qkv_kernel(x_ref, pos_ref, wq_ref, wk_ref, wv_ref, q_ref, k_ref, v_ref):
    x = x_ref[...]
    pos = pos_ref[0, :].astype(jnp.float32)
    exps = jax.lax.broadcasted_iota(jnp.int32, (1, HALF), 1).astype(
        jnp.float32) * (2.0 / HD)
    inv_freq = jnp.exp(exps * (-np.log(THETA)))
    freqs = pos[:, None] * inv_freq
    cos = jnp.cos(freqs)[:, None, :]
    sin = jnp.sin(freqs)[:, None, :]

    def rope(t, nh):
        t = t.reshape(BS, nh, HD)
        t1 = t[..., :HALF]
        t2 = t[..., HALF:]
        out = jnp.concatenate([t1 * cos - t2 * sin, t2 * cos + t1 * sin],
                              axis=-1)
        return out.reshape(BS, nh * HD)

    q = jnp.dot(x, wq_ref[...], preferred_element_type=jnp.float32)
    k = jnp.dot(x, wk_ref[...], preferred_element_type=jnp.float32)
    v = jnp.dot(x, wv_ref[...], preferred_element_type=jnp.float32)
    q_ref[...] = rope(q, NH)
    k_ref[...] = rope(k, NKV)
    v_ref[...] = v


def _attn_kernel(q_ref, k_ref, v_ref, o_ref):
    i = pl.program_id(1)
    q = q_ref[...] * SCALE
    row = jax.lax.broadcasted_iota(jnp.int32, (BQ, BK), 0) + i * BQ
    col = jax.lax.broadcasted_iota(jnp.int32, (BQ, BK), 1)

    def body(j, carry):
        acc, m, l = carry
        kb = k_ref[pl.ds(j * BK, BK), :]
        vb = v_ref[pl.ds(j * BK, BK), :]
        s = jnp.dot(q, kb.T, preferred_element_type=jnp.float32)
        s = jnp.where(col + j * BK <= row, s, -1e30)
        m_new = jnp.maximum(m, jnp.max(s, axis=-1, keepdims=True))
        p = jnp.exp(s - m_new)
        alpha = jnp.exp(m - m_new)
        l = l * alpha + jnp.sum(p, axis=-1, keepdims=True)
        acc = acc * alpha + jnp.dot(p, vb, preferred_element_type=jnp.float32)
        return acc, m_new, l

    acc = jnp.zeros((BQ, HD), jnp.float32)
    m = jnp.full((BQ, 1), -1e30, jnp.float32)
    l = jnp.zeros((BQ, 1), jnp.float32)
    acc, m, l = jax.lax.fori_loop(0, i + 1, body, (acc, m, l))
    o_ref[...] = acc / l


def _oproj_kernel(x_ref, wo_ref, o_ref):
    o_ref[...] = jnp.dot(x_ref[...], wo_ref[...],
                         preferred_element_type=jnp.float32)


def kernel(hidden_states, position_ids, Wq, Wk, Wv, Wo):
    x = hidden_states.reshape(S, HID)

    q, k, v = pl.pallas_call(
        _qkv_kernel,
        grid=(S // BS,),
        in_specs=[
            pl.BlockSpec((BS, HID), lambda i: (i, 0)),
            pl.BlockSpec((1, BS), lambda i: (0, i)),
            pl.BlockSpec((HID, NH * HD), lambda i: (0, 0)),
            pl.BlockSpec((HID, NKV * HD), lambda i: (0, 0)),
            pl.BlockSpec((HID, NKV * HD), lambda i: (0, 0)),
        ],
        out_specs=[
            pl.BlockSpec((BS, NH * HD), lambda i: (i, 0)),
            pl.BlockSpec((BS, NKV * HD), lambda i: (i, 0)),
            pl.BlockSpec((BS, NKV * HD), lambda i: (i, 0)),
        ],
        out_shape=[
            jax.ShapeDtypeStruct((S, NH * HD), jnp.float32),
            jax.ShapeDtypeStruct((S, NKV * HD), jnp.float32),
            jax.ShapeDtypeStruct((S, NKV * HD), jnp.float32),
        ],
    )(x, position_ids, Wq, Wk, Wv)

    attn = pl.pallas_call(
        _attn_kernel,
        grid=(NH, S // BQ),
        in_specs=[
            pl.BlockSpec((BQ, HD), lambda h, i: (i, h)),
            pl.BlockSpec((S, HD), lambda h, i: (0, h // N_REP)),
            pl.BlockSpec((S, HD), lambda h, i: (0, h // N_REP)),
        ],
        out_specs=pl.BlockSpec((BQ, HD), lambda h, i: (i, h)),
        out_shape=jax.ShapeDtypeStruct((S, NH * HD), jnp.float32),
    )(q, k, v)

    out = pl.pallas_call(
        _oproj_kernel,
        grid=(S // BS,),
        in_specs=[
            pl.BlockSpec((BS, NH * HD), lambda i: (i, 0)),
            pl.BlockSpec((NH * HD, HID), lambda i: (0, 0)),
        ],
        out_specs=pl.BlockSpec((BS, HID), lambda i: (i, 0)),
        out_shape=jax.ShapeDtypeStruct((S, HID), jnp.float32),
    )(attn, Wo)

    return out.reshape(B, S, HID)
